# baseline (speedup 1.0000x reference)
import functools

import jax
import jax.numpy as jnp
from jax import lax
from jax.experimental import pallas as pl
from jax.experimental.pallas import tpu as pltpu
from jax.experimental.pallas import tpu_sc as plsc

_VOCAB = 100000
_DIM = 128
_NUM_VISITS = 8192
_MAX_CODES = 64
_EPS = 1e-6

_NC = 2
_NS = 16
_NW = _NC * _NS
_VPW = _NUM_VISITS // _NW
_CODES_PW = _VPW * _MAX_CODES
_CH = 256
_NCHUNK = _CODES_PW // _CH
_VIS_PER_CH = _CH // _MAX_CODES
_NL = 16


def _augment_body(tab_ref, g_ref, gam_ref):
    z = tab_ref[...]
    zn2 = jnp.sum(z * z, axis=1, keepdims=True)
    k = (2.0 * z) / (1.0 + zn2)
    kn2 = jnp.sum(k * k, axis=1, keepdims=True)
    gamma = lax.rsqrt(jnp.clip(1.0 - kn2, _EPS, None))
    rows = lax.broadcasted_iota(jnp.int32, (z.shape[0], 1), 0) + pl.program_id(0) * z.shape[0]
    gamma = jnp.where(rows == 0, 0.0, gamma)
    gk = gamma * k
    lo = lax.bitcast_convert_type(gk[:, :_DIM // 2].astype(jnp.bfloat16),
                                  jnp.uint16).astype(jnp.int32)
    hi = lax.bitcast_convert_type(gk[:, _DIM // 2:].astype(jnp.bfloat16),
                                  jnp.uint16).astype(jnp.int32)
    g_ref[...] = lo | (hi << 16)

    kn2_1 = jnp.sum(k * k, axis=1)
    gamma_1 = lax.rsqrt(jnp.clip(1.0 - kn2_1, _EPS, None))
    rows_1 = lax.broadcasted_iota(jnp.int32, (z.shape[0],), 0) + pl.program_id(0) * z.shape[0]
    gamma_1 = jnp.where(rows_1 == 0, 0.0, gamma_1)
    gam_ref[...] = gamma_1[None, None, :]


_BA = 4000


def _augment(table):
    return pl.pallas_call(
        _augment_body,
        grid=(_VOCAB // _BA,),
        in_specs=[pl.BlockSpec((_BA, _DIM), lambda i: (i, 0))],
        out_specs=(pl.BlockSpec((_BA, _DIM // 2), lambda i: (i, 0)),
                   pl.BlockSpec((1, 1, _BA), lambda i: (i, 0, 0))),
        out_shape=(jax.ShapeDtypeStruct((_VOCAB, _DIM // 2), jnp.int32),
                   jax.ShapeDtypeStruct((_VOCAB // _BA, 1, _BA), jnp.float32)),
    )(table)


def _segsum_body(g_hbm, gam_hbm, visits_hbm, outg_hbm, outgam_hbm,
                 idx_v, bg0, bg1, bgam0, bgam1, obuf, wbuf, sem0, sem1):
    c = lax.axis_index("c")
    s = lax.axis_index("s")
    w = c * _NS + s

    pltpu.sync_copy(visits_hbm.at[w], idx_v)

    def _gather(i, bg, bgam, sem):
        pltpu.async_copy(g_hbm.at[idx_v.at[i]], bg, sem)
        pltpu.async_copy(gam_hbm.at[idx_v.at[i]], bgam, sem)

    def _wait_gather(bg, bgam, sem):
        pltpu.make_async_copy(g_hbm.at[pl.ds(0, _CH)], bg, sem).wait()
        pltpu.make_async_copy(gam_hbm.at[pl.ds(0, _CH)], bgam, sem).wait()

    def _process(i, bg, bgam):
        for v in range(_VIS_PER_CH):
            vis = i * _VIS_PER_CH + v

            def _rows(r8, accs):
                out = list(accs)
                for u in range(8):
                    row = v * _MAX_CODES + r8 * 8 + u
                    for t in range(_DIM // 2 // _NL):
                        ab = bg[row, pl.ds(t * _NL, _NL)]
                        a = lax.bitcast_convert_type(ab << 16, jnp.float32)
                        b = lax.bitcast_convert_type(ab & jnp.int32(-65536),
                                                     jnp.float32)
                        out[t] = out[t] + a
                        out[t + 4] = out[t + 4] + b
                return tuple(out)

            accs = lax.fori_loop(
                0, _MAX_CODES // 8, _rows,
                tuple(jnp.zeros((_NL,), jnp.float32) for _ in range(_DIM // _NL)))
            for t in range(_DIM // _NL):
                obuf[vis, pl.ds(t * _NL, _NL)] = accs[t]
            wsum = (bgam[pl.ds(v * _MAX_CODES, _NL)]
                    + bgam[pl.ds(v * _MAX_CODES + _NL, _NL)]
                    + bgam[pl.ds(v * _MAX_CODES + 2 * _NL, _NL)]
                    + bgam[pl.ds(v * _MAX_CODES + 3 * _NL, _NL)])
            wbuf[vis, pl.ds(0, _NL)] = wsum

    _gather(0, bg0, bgam0, sem0)

    def _two_chunks(i2, _):
        a = 2 * i2
        _gather(a + 1, bg1, bgam1, sem1)
        _wait_gather(bg0, bgam0, sem0)
        _process(a, bg0, bgam0)

        @pl.when(i2 < _NCHUNK // 2 - 1)
        def _():
            _gather(a + 2, bg0, bgam0, sem0)
        _wait_gather(bg1, bgam1, sem1)
        _process(a + 1, bg1, bgam1)
        return 0

    lax.fori_loop(0, _NCHUNK // 2, _two_chunks, 0)

    pltpu.sync_copy(obuf, outg_hbm.at[pl.ds(w * _VPW, _VPW)])
    pltpu.sync_copy(wbuf, outgam_hbm.at[pl.ds(w * _VPW, _VPW)])


@functools.cache
def _segsum():
    return pl.kernel(
        _segsum_body,
        out_type=(jax.ShapeDtypeStruct((_NUM_VISITS, _DIM), jnp.float32),
                  jax.ShapeDtypeStruct((_NUM_VISITS, _NL), jnp.float32)),
        mesh=plsc.VectorSubcoreMesh(core_axis_name="c", subcore_axis_name="s",
                                    num_cores=_NC, num_subcores=_NS),
        compiler_params=pltpu.CompilerParams(use_tc_tiling_on_sc=False),
        scratch_types=[
            pltpu.VMEM((_NCHUNK, _CH), jnp.int32),
            pltpu.VMEM((_CH, _DIM // 2), jnp.int32),
            pltpu.VMEM((_CH, _DIM // 2), jnp.int32),
            pltpu.VMEM((_CH,), jnp.float32),
            pltpu.VMEM((_CH,), jnp.float32),
            pltpu.VMEM((_VPW, _DIM), jnp.float32),
            pltpu.VMEM((_VPW, _NL), jnp.float32),
            pltpu.SemaphoreType.DMA,
            pltpu.SemaphoreType.DMA,
        ],
    )


def _finish_body(sg_ref, sgam_ref, o_ref):
    S = sg_ref[...]
    W = jnp.sum(sgam_ref[...], axis=1, keepdims=True)
    m = S / jnp.clip(W, _EPS, None)
    mn2 = jnp.sum(m * m, axis=1, keepdims=True)
    p = m / (1.0 + jnp.sqrt(jnp.clip(1.0 - mn2, _EPS, None)))
    pn = jnp.sqrt(jnp.clip(jnp.sum(p * p, axis=1, keepdims=True), _EPS, None))
    pc = jnp.clip(pn, None, 1.0 - 1e-5)
    o_ref[...] = (0.5 * jnp.log((1.0 + pc) / (1.0 - pc))) * p / pn


_BC = 512


def _finish(sums_g, sums_gam):
    return pl.pallas_call(
        _finish_body,
        grid=(_NUM_VISITS // _BC,),
        in_specs=[pl.BlockSpec((_BC, _DIM), lambda i: (i, 0)),
                  pl.BlockSpec((_BC, _NL), lambda i: (i, 0))],
        out_specs=pl.BlockSpec((_BC, _DIM), lambda i: (i, 0)),
        out_shape=jax.ShapeDtypeStruct((_NUM_VISITS, _DIM), jnp.float32),
    )(sums_g, sums_gam)


def kernel(visits, table):
    g_tbl, gam2 = _augment(table)
    gam_tbl = gam2.reshape(_VOCAB)
    visits3 = visits.reshape(_NW, _NCHUNK, _CH)
    sums_g, sums_gam = _segsum()(g_tbl, gam_tbl, visits3)
    return _finish(sums_g, sums_gam)


# speedup vs baseline: 1.1987x; 1.1987x over previous
import functools

import jax
import jax.numpy as jnp
from jax import lax
from jax.experimental import pallas as pl
from jax.experimental.pallas import tpu as pltpu
from jax.experimental.pallas import tpu_sc as plsc

_VOCAB = 100000
_DIM = 128
_NUM_VISITS = 8192
_MAX_CODES = 64
_EPS = 1e-6

_NC = 2
_NS = 16
_NW = _NC * _NS
_VPW = _NUM_VISITS // _NW
_CODES_PW = _VPW * _MAX_CODES
_CH = 256
_NCHUNK = _CODES_PW // _CH
_VIS_PER_CH = _CH // _MAX_CODES
_NL = 16


_BH = 2000


def _augment_half(z, row_base):
    zn2 = jnp.sum(z * z, axis=1, keepdims=True)
    k = (2.0 * z) / (1.0 + zn2)
    kn2 = jnp.sum(k * k, axis=1, keepdims=True)
    gamma = lax.rsqrt(jnp.clip(1.0 - kn2, _EPS, None))
    rows = lax.broadcasted_iota(jnp.int32, (z.shape[0], 1), 0) + row_base
    gamma = jnp.where(rows == 0, 0.0, gamma)
    gk = gamma * k
    lo = lax.bitcast_convert_type(gk[:, :_DIM // 2].astype(jnp.bfloat16),
                                  jnp.uint16).astype(jnp.int32)
    hi = lax.bitcast_convert_type(gk[:, _DIM // 2:].astype(jnp.bfloat16),
                                  jnp.uint16).astype(jnp.int32)
    packed = lo | (hi << 16)

    kn2_1 = jnp.sum(k * k, axis=1)
    gamma_1 = lax.rsqrt(jnp.clip(1.0 - kn2_1, _EPS, None))
    rows_1 = lax.broadcasted_iota(jnp.int32, (z.shape[0],), 0) + row_base
    gamma_1 = jnp.where(rows_1 == 0, 0.0, gamma_1)
    return packed, gamma_1


def _augment_body(lo_ref, hi_ref, g_ref, gam_ref):
    base = pl.program_id(0) * _BH
    p_lo, g_lo = _augment_half(lo_ref[...], base)
    p_hi, g_hi = _augment_half(hi_ref[...], _VOCAB // 2 + base)
    g_ref[...] = jnp.concatenate([p_lo, p_hi], axis=1)
    gam_ref[...] = jnp.concatenate([g_lo, g_hi], axis=0)[None, None, :]


def _augment(table):
    return pl.pallas_call(
        _augment_body,
        grid=(_VOCAB // 2 // _BH,),
        in_specs=[pl.BlockSpec((_BH, _DIM), lambda i: (i, 0)),
                  pl.BlockSpec((_BH, _DIM), lambda i: (i + _VOCAB // 2 // _BH, 0))],
        out_specs=(pl.BlockSpec((_BH, _DIM), lambda i: (i, 0)),
                   pl.BlockSpec((1, 1, 2 * _BH), lambda i: (i, 0, 0))),
        out_shape=(jax.ShapeDtypeStruct((_VOCAB // 2, _DIM), jnp.int32),
                   jax.ShapeDtypeStruct((_VOCAB // 2 // _BH, 1, 2 * _BH), jnp.float32)),
    )(table, table)


def _segsum_body(g_hbm, gam_hbm, visits_hbm, outg_hbm, outgam_hbm,
                 idx_v, idxg_v, bg0, bg1, bgam0, bgam1, obuf, wbuf, sem0, sem1):
    c = lax.axis_index("c")
    s = lax.axis_index("s")
    w = c * _NS + s

    pltpu.sync_copy(visits_hbm.at[w], idx_v)

    def _remap(rr, _):
        for gg in range(_CH // _NL):
            r = idx_v[rr, pl.ds(gg * _NL, _NL)]
            h = jnp.where(r >= _VOCAB // 2, 1, 0).astype(jnp.int32)
            m = r - h * (_VOCAB // 2)
            ii = (m * 33555) >> 26
            u = m - ii * _BH
            idxg_v[rr, pl.ds(gg * _NL, _NL)] = 2 * _BH * ii + 2 * u + h
            idx_v[rr, pl.ds(gg * _NL, _NL)] = 2 * _BH * ii + _BH * h + u
        return 0
    lax.fori_loop(0, _NCHUNK, _remap, 0)

    def _gather(i, bg, bgam, sem):
        pltpu.async_copy(g_hbm.at[idxg_v.at[i]], bg, sem)
        pltpu.async_copy(gam_hbm.at[idx_v.at[i]], bgam, sem)

    def _wait_gather(bg, bgam, sem):
        pltpu.make_async_copy(g_hbm.at[pl.ds(0, _CH)], bg, sem).wait()
        pltpu.make_async_copy(gam_hbm.at[pl.ds(0, _CH)], bgam, sem).wait()

    def _process(i, bg, bgam):
        for v in range(_VIS_PER_CH):
            vis = i * _VIS_PER_CH + v

            def _rows(r8, accs):
                out = list(accs)
                for u in range(8):
                    row = v * _MAX_CODES + r8 * 8 + u
                    for t in range(_DIM // 2 // _NL):
                        ab = bg[row, pl.ds(t * _NL, _NL)]
                        a = lax.bitcast_convert_type(ab << 16, jnp.float32)
                        b = lax.bitcast_convert_type(ab & jnp.int32(-65536),
                                                     jnp.float32)
                        out[t] = out[t] + a
                        out[t + 4] = out[t + 4] + b
                return tuple(out)

            accs = lax.fori_loop(
                0, _MAX_CODES // 8, _rows,
                tuple(jnp.zeros((_NL,), jnp.float32) for _ in range(_DIM // _NL)))
            for t in range(_DIM // _NL):
                obuf[vis, pl.ds(t * _NL, _NL)] = accs[t]
            wsum = (bgam[pl.ds(v * _MAX_CODES, _NL)]
                    + bgam[pl.ds(v * _MAX_CODES + _NL, _NL)]
                    + bgam[pl.ds(v * _MAX_CODES + 2 * _NL, _NL)]
                    + bgam[pl.ds(v * _MAX_CODES + 3 * _NL, _NL)])
            wbuf[vis, pl.ds(0, _NL)] = wsum

    _gather(0, bg0, bgam0, sem0)

    def _two_chunks(i2, _):
        a = 2 * i2
        _gather(a + 1, bg1, bgam1, sem1)
        _wait_gather(bg0, bgam0, sem0)
        _process(a, bg0, bgam0)

        @pl.when(i2 < _NCHUNK // 2 - 1)
        def _():
            _gather(a + 2, bg0, bgam0, sem0)
        _wait_gather(bg1, bgam1, sem1)
        _process(a + 1, bg1, bgam1)
        return 0

    lax.fori_loop(0, _NCHUNK // 2, _two_chunks, 0)

    pltpu.sync_copy(obuf, outg_hbm.at[pl.ds(w * _VPW, _VPW)])
    pltpu.sync_copy(wbuf, outgam_hbm.at[pl.ds(w * _VPW, _VPW)])


@functools.cache
def _segsum():
    return pl.kernel(
        _segsum_body,
        out_type=(jax.ShapeDtypeStruct((_NUM_VISITS, _DIM), jnp.float32),
                  jax.ShapeDtypeStruct((_NUM_VISITS, _NL), jnp.float32)),
        mesh=plsc.VectorSubcoreMesh(core_axis_name="c", subcore_axis_name="s",
                                    num_cores=_NC, num_subcores=_NS),
        compiler_params=pltpu.CompilerParams(use_tc_tiling_on_sc=False),
        scratch_types=[
            pltpu.VMEM((_NCHUNK, _CH), jnp.int32),
            pltpu.VMEM((_NCHUNK, _CH), jnp.int32),
            pltpu.VMEM((_CH, _DIM // 2), jnp.int32),
            pltpu.VMEM((_CH, _DIM // 2), jnp.int32),
            pltpu.VMEM((_CH,), jnp.float32),
            pltpu.VMEM((_CH,), jnp.float32),
            pltpu.VMEM((_VPW, _DIM), jnp.float32),
            pltpu.VMEM((_VPW, _NL), jnp.float32),
            pltpu.SemaphoreType.DMA,
            pltpu.SemaphoreType.DMA,
        ],
    )


def _finish_body(sg_ref, sgam_ref, o_ref):
    S = sg_ref[...]
    W = jnp.sum(sgam_ref[...], axis=1, keepdims=True)
    m = S / jnp.clip(W, _EPS, None)
    mn2 = jnp.sum(m * m, axis=1, keepdims=True)
    p = m / (1.0 + jnp.sqrt(jnp.clip(1.0 - mn2, _EPS, None)))
    pn = jnp.sqrt(jnp.clip(jnp.sum(p * p, axis=1, keepdims=True), _EPS, None))
    pc = jnp.clip(pn, None, 1.0 - 1e-5)
    o_ref[...] = (0.5 * jnp.log((1.0 + pc) / (1.0 - pc))) * p / pn


_BC = 512


def _finish(sums_g, sums_gam):
    return pl.pallas_call(
        _finish_body,
        grid=(_NUM_VISITS // _BC,),
        in_specs=[pl.BlockSpec((_BC, _DIM), lambda i: (i, 0)),
                  pl.BlockSpec((_BC, _NL), lambda i: (i, 0))],
        out_specs=pl.BlockSpec((_BC, _DIM), lambda i: (i, 0)),
        out_shape=jax.ShapeDtypeStruct((_NUM_VISITS, _DIM), jnp.float32),
    )(sums_g, sums_gam)


def kernel(visits, table):
    g2, gam2 = _augment(table)
    g_tbl = g2.reshape(_VOCAB, _DIM // 2)
    gam_tbl = gam2.reshape(_VOCAB)
    visits3 = visits.reshape(_NW, _NCHUNK, _CH)
    sums_g, sums_gam = _segsum()(g_tbl, gam_tbl, visits3)
    return _finish(sums_g, sums_gam)


# speedup vs baseline: 1.2356x; 1.0307x over previous
import functools

import jax
import jax.numpy as jnp
from jax import lax
from jax.experimental import pallas as pl
from jax.experimental.pallas import tpu as pltpu
from jax.experimental.pallas import tpu_sc as plsc

_VOCAB = 100000
_DIM = 128
_NUM_VISITS = 8192
_MAX_CODES = 64
_EPS = 1e-6

_NC = 2
_NS = 16
_NW = _NC * _NS
_VPW = _NUM_VISITS // _NW
_CODES_PW = _VPW * _MAX_CODES
_CH = 256
_NCHUNK = _CODES_PW // _CH
_VIS_PER_CH = _CH // _MAX_CODES
_NL = 16


_BH = 2000
_BAH = 5000


def _augment_half(z, row_base):
    zn2 = jnp.sum(z * z, axis=1, keepdims=True)
    k = (2.0 * z) / (1.0 + zn2)
    kn2 = jnp.sum(k * k, axis=1, keepdims=True)
    gamma = lax.rsqrt(jnp.clip(1.0 - kn2, _EPS, None))
    rows = lax.broadcasted_iota(jnp.int32, (z.shape[0], 1), 0) + row_base
    gamma = jnp.where(rows == 0, 0.0, gamma)
    gk = gamma * k
    lo = lax.bitcast_convert_type(gk[:, :_DIM // 2].astype(jnp.bfloat16),
                                  jnp.uint16).astype(jnp.int32)
    hi = lax.bitcast_convert_type(gk[:, _DIM // 2:].astype(jnp.bfloat16),
                                  jnp.uint16).astype(jnp.int32)
    packed = lo | (hi << 16)

    kn2_1 = jnp.sum(k * k, axis=1)
    gamma_1 = lax.rsqrt(jnp.clip(1.0 - kn2_1, _EPS, None))
    rows_1 = lax.broadcasted_iota(jnp.int32, (z.shape[0],), 0) + row_base
    gamma_1 = jnp.where(rows_1 == 0, 0.0, gamma_1)
    return packed, gamma_1


def _augment_body(lo_ref, hi_ref, g_ref, gam_ref):
    base = pl.program_id(0) * _BAH
    p_lo, g_lo = _augment_half(lo_ref[...], base)
    p_hi, g_hi = _augment_half(hi_ref[...], _VOCAB // 2 + base)
    g_ref[...] = jnp.concatenate([p_lo, p_hi], axis=1)
    gam_ref[...] = jnp.concatenate([g_lo, g_hi], axis=0)[None, None, :]


def _augment(table):
    return pl.pallas_call(
        _augment_body,
        grid=(_VOCAB // 2 // _BAH,),
        in_specs=[pl.BlockSpec((_BAH, _DIM), lambda i: (i, 0)),
                  pl.BlockSpec((_BAH, _DIM), lambda i: (i + _VOCAB // 2 // _BAH, 0))],
        out_specs=(pl.BlockSpec((_BAH, _DIM), lambda i: (i, 0)),
                   pl.BlockSpec((1, 1, 2 * _BAH), lambda i: (i, 0, 0))),
        out_shape=(jax.ShapeDtypeStruct((_VOCAB // 2, _DIM), jnp.int32),
                   jax.ShapeDtypeStruct((_VOCAB // 2 // _BAH, 1, 2 * _BAH), jnp.float32)),
    )(table, table)


def _segsum_body(g_hbm, gam_hbm, visits_hbm, outg_hbm, outgam_hbm,
                 idx_v, idxg_v, bg0, bg1, bgam_all, obuf, wbuf,
                 sem0, sem1, semg):
    c = lax.axis_index("c")
    s = lax.axis_index("s")
    w = c * _NS + s

    pltpu.sync_copy(visits_hbm.at[w], idx_v)

    def _remap(rr, _):
        for gg in range(_CH // _NL):
            r = idx_v[rr, pl.ds(gg * _NL, _NL)]
            h = jnp.where(r >= _VOCAB // 2, 1, 0).astype(jnp.int32)
            m = r - h * (_VOCAB // 2)
            ii = (m * 13422) >> 26
            u = m - ii * _BAH
            idxg_v[rr, pl.ds(gg * _NL, _NL)] = 2 * m + h
            idx_v[rr, pl.ds(gg * _NL, _NL)] = 2 * _BAH * ii + _BAH * h + u
        return 0
    lax.fori_loop(0, _NCHUNK, _remap, 0)

    def _gather_gam(i):
        pltpu.async_copy(gam_hbm.at[idx_v.at[i]], bgam_all.at[i], semg)

    def _gather(i, bg, sem):
        pltpu.async_copy(g_hbm.at[idxg_v.at[i]], bg, sem)

    def _wait_gather(bg, sem):
        pltpu.make_async_copy(g_hbm.at[pl.ds(0, _CH)], bg, sem).wait()

    def _process(i, bg):
        for v in range(_VIS_PER_CH):
            vis = i * _VIS_PER_CH + v

            def _rows(r8, accs):
                out = list(accs)
                for u in range(8):
                    row = v * _MAX_CODES + r8 * 8 + u
                    for t in range(_DIM // 2 // _NL):
                        ab = bg[row, pl.ds(t * _NL, _NL)]
                        a = lax.bitcast_convert_type(ab << 16, jnp.float32)
                        b = lax.bitcast_convert_type(ab & jnp.int32(-65536),
                                                     jnp.float32)
                        out[t] = out[t] + a
                        out[t + 4] = out[t + 4] + b
                return tuple(out)

            accs = lax.fori_loop(
                0, _MAX_CODES // 8, _rows,
                tuple(jnp.zeros((_NL,), jnp.float32) for _ in range(_DIM // _NL)))
            for t in range(_DIM // _NL):
                obuf[vis, pl.ds(t * _NL, _NL)] = accs[t]

    _gather(0, bg0, sem0)

    def _two_chunks(i2, _):
        a = 2 * i2
        _gather(a + 1, bg1, sem1)
        _gather_gam(a)
        _gather_gam(a + 1)
        _wait_gather(bg0, sem0)
        _process(a, bg0)

        @pl.when(i2 < _NCHUNK // 2 - 1)
        def _():
            _gather(a + 2, bg0, sem0)
        _wait_gather(bg1, sem1)
        _process(a + 1, bg1)
        return 0

    lax.fori_loop(0, _NCHUNK // 2, _two_chunks, 0)

    def _drain_gam(i, _):
        pltpu.make_async_copy(gam_hbm.at[idx_v.at[i]], bgam_all.at[i],
                              semg).wait()
        return 0
    lax.fori_loop(0, _NCHUNK, _drain_gam, 0)

    def _wpass(i, _):
        for v in range(_VIS_PER_CH):
            wsum = (bgam_all[i, pl.ds(v * _MAX_CODES, _NL)]
                    + bgam_all[i, pl.ds(v * _MAX_CODES + _NL, _NL)]
                    + bgam_all[i, pl.ds(v * _MAX_CODES + 2 * _NL, _NL)]
                    + bgam_all[i, pl.ds(v * _MAX_CODES + 3 * _NL, _NL)])
            wbuf[i * _VIS_PER_CH + v, pl.ds(0, _NL)] = wsum
        return 0

    lax.fori_loop(0, _NCHUNK, _wpass, 0)

    pltpu.sync_copy(obuf, outg_hbm.at[pl.ds(w * _VPW, _VPW)])
    pltpu.sync_copy(wbuf, outgam_hbm.at[pl.ds(w * _VPW, _VPW)])


@functools.cache
def _segsum():
    return pl.kernel(
        _segsum_body,
        out_type=(jax.ShapeDtypeStruct((_NUM_VISITS, _DIM), jnp.float32),
                  jax.ShapeDtypeStruct((_NUM_VISITS, _NL), jnp.float32)),
        mesh=plsc.VectorSubcoreMesh(core_axis_name="c", subcore_axis_name="s",
                                    num_cores=_NC, num_subcores=_NS),
        compiler_params=pltpu.CompilerParams(use_tc_tiling_on_sc=False),
        scratch_types=[
            pltpu.VMEM((_NCHUNK, _CH), jnp.int32),
            pltpu.VMEM((_NCHUNK, _CH), jnp.int32),
            pltpu.VMEM((_CH, _DIM // 2), jnp.int32),
            pltpu.VMEM((_CH, _DIM // 2), jnp.int32),
            pltpu.VMEM((_NCHUNK, _CH), jnp.float32),
            pltpu.VMEM((_VPW, _DIM), jnp.float32),
            pltpu.VMEM((_VPW, _NL), jnp.float32),
            pltpu.SemaphoreType.DMA,
            pltpu.SemaphoreType.DMA,
            pltpu.SemaphoreType.DMA,
        ],
    )


def _finish_body(sg_ref, sgam_ref, o_ref):
    S = sg_ref[...]
    W = jnp.sum(sgam_ref[...], axis=1, keepdims=True)
    m = S / jnp.clip(W, _EPS, None)
    mn2 = jnp.sum(m * m, axis=1, keepdims=True)
    p = m / (1.0 + jnp.sqrt(jnp.clip(1.0 - mn2, _EPS, None)))
    pn = jnp.sqrt(jnp.clip(jnp.sum(p * p, axis=1, keepdims=True), _EPS, None))
    pc = jnp.clip(pn, None, 1.0 - 1e-5)
    o_ref[...] = (0.5 * jnp.log((1.0 + pc) / (1.0 - pc))) * p / pn


_BC = 512


def _finish(sums_g, sums_gam):
    return pl.pallas_call(
        _finish_body,
        grid=(_NUM_VISITS // _BC,),
        in_specs=[pl.BlockSpec((_BC, _DIM), lambda i: (i, 0)),
                  pl.BlockSpec((_BC, _NL), lambda i: (i, 0))],
        out_specs=pl.BlockSpec((_BC, _DIM), lambda i: (i, 0)),
        out_shape=jax.ShapeDtypeStruct((_NUM_VISITS, _DIM), jnp.float32),
    )(sums_g, sums_gam)


def kernel(visits, table):
    g2, gam2 = _augment(table)
    g_tbl = g2.reshape(_VOCAB, _DIM // 2)
    gam_tbl = gam2.reshape(_VOCAB)
    visits3 = visits.reshape(_NW, _NCHUNK, _CH)
    sums_g, sums_gam = _segsum()(g_tbl, gam_tbl, visits3)
    return _finish(sums_g, sums_gam)


# speedup vs baseline: 1.2391x; 1.0028x over previous
import functools

import jax
import jax.numpy as jnp
from jax import lax
from jax.experimental import pallas as pl
from jax.experimental.pallas import tpu as pltpu
from jax.experimental.pallas import tpu_sc as plsc

_VOCAB = 100000
_DIM = 128
_NUM_VISITS = 8192
_MAX_CODES = 64
_EPS = 1e-6

_NC = 2
_NS = 16
_NW = _NC * _NS
_VPW = _NUM_VISITS // _NW
_CODES_PW = _VPW * _MAX_CODES
_CH = 256
_NCHUNK = _CODES_PW // _CH
_VIS_PER_CH = _CH // _MAX_CODES
_NL = 16


_BH = 2000
_BAH = 5000


def _augment_half(z, row_base):
    zn2 = jnp.sum(z * z, axis=1, keepdims=True)
    k = (2.0 * z) / (1.0 + zn2)
    kn2 = jnp.sum(k * k, axis=1, keepdims=True)
    gamma = lax.rsqrt(jnp.clip(1.0 - kn2, _EPS, None))
    rows = lax.broadcasted_iota(jnp.int32, (z.shape[0], 1), 0) + row_base
    gamma = jnp.where(rows == 0, 0.0, gamma)
    gk = gamma * k
    lo = lax.bitcast_convert_type(gk[:, :_DIM // 2].astype(jnp.bfloat16),
                                  jnp.uint16).astype(jnp.int32)
    hi = lax.bitcast_convert_type(gk[:, _DIM // 2:].astype(jnp.bfloat16),
                                  jnp.uint16).astype(jnp.int32)
    packed = lo | (hi << 16)

    kn2_1 = jnp.sum(k * k, axis=1)
    gamma_1 = lax.rsqrt(jnp.clip(1.0 - kn2_1, _EPS, None))
    rows_1 = lax.broadcasted_iota(jnp.int32, (z.shape[0],), 0) + row_base
    gamma_1 = jnp.where(rows_1 == 0, 0.0, gamma_1)
    return packed, gamma_1


def _augment_body(lo_ref, hi_ref, g_ref, gam_ref):
    base = pl.program_id(0) * _BAH
    p_lo, g_lo = _augment_half(lo_ref[...], base)
    p_hi, g_hi = _augment_half(hi_ref[...], _VOCAB // 2 + base)
    g_ref[...] = jnp.concatenate([p_lo, p_hi], axis=1)
    gam_ref[...] = jnp.concatenate([g_lo, g_hi], axis=0)[None, None, :]


def _augment(table):
    return pl.pallas_call(
        _augment_body,
        grid=(_VOCAB // 2 // _BAH,),
        in_specs=[pl.BlockSpec((_BAH, _DIM), lambda i: (i, 0)),
                  pl.BlockSpec((_BAH, _DIM), lambda i: (i + _VOCAB // 2 // _BAH, 0))],
        out_specs=(pl.BlockSpec((_BAH, _DIM), lambda i: (i, 0)),
                   pl.BlockSpec((1, 1, 2 * _BAH), lambda i: (i, 0, 0))),
        out_shape=(jax.ShapeDtypeStruct((_VOCAB // 2, _DIM), jnp.int32),
                   jax.ShapeDtypeStruct((_VOCAB // 2 // _BAH, 1, 2 * _BAH), jnp.float32)),
    )(table, table)


def _segsum_body(g_hbm, gam_hbm, visits_hbm, outg_hbm, outgam_hbm,
                 idx_v, idxg_v, bg0, bg1, bgam_all, obuf, wbuf,
                 sem0, sem1, semg):
    c = lax.axis_index("c")
    s = lax.axis_index("s")
    w = c * _NS + s

    pltpu.sync_copy(visits_hbm.at[w], idx_v)

    def _remap(rr, _):
        for gg in range(_CH // _NL):
            r = idx_v[rr, pl.ds(gg * _NL, _NL)]
            h = jnp.where(r >= _VOCAB // 2, 1, 0).astype(jnp.int32)
            m = r - h * (_VOCAB // 2)
            ii = (m * 13422) >> 26
            u = m - ii * _BAH
            idxg_v[rr, pl.ds(gg * _NL, _NL)] = 2 * m + h
            idx_v[rr, pl.ds(gg * _NL, _NL)] = 2 * _BAH * ii + _BAH * h + u
        return 0
    lax.fori_loop(0, _NCHUNK, _remap, 0)

    def _gather_gam(i):
        pltpu.async_copy(gam_hbm.at[idx_v.at[i]], bgam_all.at[i], semg)

    def _gather(i, bg, sem):
        pltpu.async_copy(g_hbm.at[idxg_v.at[i]], bg, sem)

    def _wait_gather(bg, sem):
        pltpu.make_async_copy(g_hbm.at[pl.ds(0, _CH)], bg, sem).wait()

    def _process(i, bg):
        for v in range(_VIS_PER_CH):
            vis = i * _VIS_PER_CH + v

            def _rows(r8, accs):
                out = list(accs)
                for u in range(8):
                    row = v * _MAX_CODES + r8 * 8 + u
                    for t in range(_DIM // 2 // _NL):
                        ab = bg[row, pl.ds(t * _NL, _NL)]
                        a = lax.bitcast_convert_type(ab << 16, jnp.float32)
                        b = lax.bitcast_convert_type(ab & jnp.int32(-65536),
                                                     jnp.float32)
                        out[t] = out[t] + a
                        out[t + 4] = out[t + 4] + b
                return tuple(out)

            accs = lax.fori_loop(
                0, _MAX_CODES // 8, _rows,
                tuple(jnp.zeros((_NL,), jnp.float32) for _ in range(_DIM // _NL)))
            for t in range(_DIM // _NL):
                obuf[vis, pl.ds(t * _NL, _NL)] = accs[t]

    _gather(0, bg0, sem0)

    def _two_chunks(i2, _):
        a = 2 * i2
        _gather(a + 1, bg1, sem1)
        _gather_gam(a)
        _gather_gam(a + 1)
        _wait_gather(bg0, sem0)
        _process(a, bg0)

        @pl.when(i2 < _NCHUNK // 2 - 1)
        def _():
            _gather(a + 2, bg0, sem0)
        _wait_gather(bg1, sem1)
        _process(a + 1, bg1)
        return 0

    lax.fori_loop(0, _NCHUNK // 2, _two_chunks, 0)

    def _drain_gam(i, _):
        pltpu.make_async_copy(gam_hbm.at[idx_v.at[i]], bgam_all.at[i],
                              semg).wait()
        return 0
    lax.fori_loop(0, _NCHUNK, _drain_gam, 0)

    def _wpass(i, _):
        for v in range(_VIS_PER_CH):
            wsum = (bgam_all[i, pl.ds(v * _MAX_CODES, _NL)]
                    + bgam_all[i, pl.ds(v * _MAX_CODES + _NL, _NL)]
                    + bgam_all[i, pl.ds(v * _MAX_CODES + 2 * _NL, _NL)]
                    + bgam_all[i, pl.ds(v * _MAX_CODES + 3 * _NL, _NL)])
            wbuf[i * _VIS_PER_CH + v, pl.ds(0, _NL)] = wsum
        return 0

    lax.fori_loop(0, _NCHUNK, _wpass, 0)

    pltpu.sync_copy(obuf, outg_hbm.at[pl.ds(w * _VPW, _VPW)])
    pltpu.sync_copy(wbuf, outgam_hbm.at[pl.ds(w * _VPW, _VPW)])


@functools.cache
def _segsum():
    return pl.kernel(
        _segsum_body,
        out_type=(jax.ShapeDtypeStruct((_NUM_VISITS, _DIM), jnp.float32),
                  jax.ShapeDtypeStruct((_NUM_VISITS, _NL), jnp.float32)),
        mesh=plsc.VectorSubcoreMesh(core_axis_name="c", subcore_axis_name="s",
                                    num_cores=_NC, num_subcores=_NS),
        compiler_params=pltpu.CompilerParams(use_tc_tiling_on_sc=False),
        scratch_types=[
            pltpu.VMEM((_NCHUNK, _CH), jnp.int32),
            pltpu.VMEM((_NCHUNK, _CH), jnp.int32),
            pltpu.VMEM((_CH, _DIM // 2), jnp.int32),
            pltpu.VMEM((_CH, _DIM // 2), jnp.int32),
            pltpu.VMEM((_NCHUNK, _CH), jnp.float32),
            pltpu.VMEM((_VPW, _DIM), jnp.float32),
            pltpu.VMEM((_VPW, _NL), jnp.float32),
            pltpu.SemaphoreType.DMA,
            pltpu.SemaphoreType.DMA,
            pltpu.SemaphoreType.DMA,
        ],
    )


def _finish_body(sg_ref, sgam_ref, o_ref):
    S = sg_ref[...]
    W = jnp.sum(sgam_ref[...], axis=1, keepdims=True)
    sn2 = jnp.sum(S * S, axis=1, keepdims=True)
    rw = 1.0 / jnp.clip(W, _EPS, None)
    mn2 = sn2 * rw * rw
    rq = rw / (1.0 + jnp.sqrt(jnp.clip(1.0 - mn2, _EPS, None)))
    pn2 = jnp.clip(sn2 * rq * rq, _EPS, None)
    pn = jnp.sqrt(pn2)
    pc = jnp.clip(pn, None, 1.0 - 1e-5)
    scale = (0.5 * jnp.log((1.0 + pc) / (1.0 - pc))) / pn * rq
    o_ref[...] = S * scale


_BC = 512


def _finish(sums_g, sums_gam):
    return pl.pallas_call(
        _finish_body,
        grid=(_NUM_VISITS // _BC,),
        in_specs=[pl.BlockSpec((_BC, _DIM), lambda i: (i, 0)),
                  pl.BlockSpec((_BC, _NL), lambda i: (i, 0))],
        out_specs=pl.BlockSpec((_BC, _DIM), lambda i: (i, 0)),
        out_shape=jax.ShapeDtypeStruct((_NUM_VISITS, _DIM), jnp.float32),
    )(sums_g, sums_gam)


def kernel(visits, table):
    g2, gam2 = _augment(table)
    g_tbl = g2.reshape(_VOCAB, _DIM // 2)
    gam_tbl = gam2.reshape(_VOCAB)
    visits3 = visits.reshape(_NW, _NCHUNK, _CH)
    sums_g, sums_gam = _segsum()(g_tbl, gam_tbl, visits3)
    return _finish(sums_g, sums_gam)
